# baseline (speedup 1.0000x reference)
import functools

import jax
import jax.numpy as jnp
import numpy as np
from jax import lax
from jax.experimental import pallas as pl
from jax.experimental.pallas import tpu as pltpu
from jax.experimental.pallas import tpu_sc as plsc

F = 26
DIM = 128
M = 8
K = 256
B = 4096
PLEN = DIM // M
TOTAL = 100000 * F
EPS = 1e-5

NSC = 2
NSUB = 16
NW = NSC * NSUB
QROWS = B // 4
CHUNK = 64
NCHUNK = QROWS // CHUNK
CROWS = CHUNK * F // 128
LROWS = B // NW

_mesh = plsc.VectorSubcoreMesh(core_axis_name="c", subcore_axis_name="s")


NMAIN = (TOTAL // 128) * 128
NTAIL = TOTAL - NMAIN
WMAX = NMAIN * M


def _fire_gathers(cb_h, jidxb, codesb, sem):
    return [
        pltpu.async_copy(cb_h.at[jidxb.at[i]],
                         codesb.at[pl.ds(i * 128, 128)], sem)
        for i in range(CROWS)
    ]


def _sc_fm_body(gidx_h, cb_h, cbtail_h, tbl_h, cross_h,
                tblv, idxvs, jidxvs, codesvs, tailv, outvs,
                gsems, isems, osems):
    ci = lax.axis_index("c")
    si = lax.axis_index("s")
    wid = si * NSC + ci
    j = lax.rem(wid, M)
    q = wid // M
    j16 = j * PLEN

    pltpu.sync_copy(tbl_h.at[j], tblv)
    pltpu.sync_copy(cbtail_h, tailv)

    iota = lax.iota(jnp.int32, 16)
    zeros_i = jnp.zeros((16,), jnp.int32)
    j128 = zeros_i + j * 128
    jv = zeros_i + j
    rbase = q * (QROWS * F // 128)

    def fire_idx(c, buf):
        pltpu.async_copy(gidx_h.at[pl.ds(rbase + c * CROWS, CROWS)],
                         idxvs[buf], isems[buf])

    def wait_idx(buf):
        pltpu.make_async_copy(gidx_h.at[pl.ds(rbase, CROWS)],
                              idxvs[buf], isems[buf]).wait()

    def fire_gathers(buf):
        for r in range(CROWS):
            for l in range(8):
                s = pl.ds(l * 16, 16)
                w = idxvs[buf][r, s] + j128
                jidxvs[buf][r, s] = jnp.minimum(w, WMAX - 1)
        _fire_gathers(cb_h, jidxvs[buf], codesvs[buf], gsems[buf])

    def pre(c, buf):
        for i in range(CROWS):
            pltpu.make_async_copy(cb_h.at[jidxvs[buf].at[i]],
                                  codesvs[buf].at[pl.ds(i * 128, 128)],
                                  gsems[buf]).wait()
        idxb, codesb = idxvs[buf], codesvs[buf]
        for gb in range(CHUNK // 16):
            p25 = (gb * 16 + iota) * F + (F - 1)
            pb = plsc.load_gather(
                idxb, [lax.shift_right_logical(p25, 7),
                       lax.bitwise_and(p25, 127)])
            is_tail = pb >= WMAX
            tv = plsc.load_gather(
                tailv, [jnp.clip(pb - WMAX, 0, NTAIL - 1) * M + jv])
            cur = plsc.load_gather(codesb, [p25])
            plsc.store_scatter(codesb, [p25], jnp.where(is_tail, tv, cur))

    def main(c, buf):
        codesb, outv = codesvs[buf], outvs[buf]

        @pl.when(c >= 2)
        def _():
            pltpu.make_async_copy(
                outvs[buf], cross_h.at[pl.ds(0, CHUNK), pl.ds(j16, PLEN)],
                osems[buf]).wait()

        @plsc.parallel_loop(0, CHUNK // 16)
        def _grp(gb):
            rowbase = (gb * 16 + iota) * F
            orow = gb * 16 + iota
            for h in range(2):
                acc_s = [jnp.zeros((16,), jnp.float32) for _ in range(8)]
                acc_t = [jnp.zeros((16,), jnp.float32) for _ in range(8)]
                for f in range(F):
                    codes = plsc.load_gather(codesb, [rowbase + f])
                    ch = (lax.shift_left(
                        lax.shift_right_logical(codes, 7), 10)
                        + lax.bitwise_and(codes, 127))
                    for di in range(8):
                        dd = h * 8 + di
                        k1 = (((dd >> 3) * 52 + f * 2) * 1024
                              + (dd & 7) * 128)
                        v = plsc.load_gather(tblv, [ch + k1])
                        acc_s[di] = acc_s[di] + v
                        acc_t[di] = acc_t[di] + v * v
                for di in range(8):
                    dd = h * 8 + di
                    cv = 0.5 * (acc_s[di] * acc_s[di] - acc_t[di])
                    plsc.store_scatter(outv, [orow, zeros_i + dd], cv)

        b0 = q * QROWS + c * CHUNK
        pltpu.async_copy(outv, cross_h.at[pl.ds(b0, CHUNK), pl.ds(j16, PLEN)],
                         osems[buf])

    fire_idx(0, 0)
    fire_idx(1, 1)
    wait_idx(0)
    fire_gathers(0)
    wait_idx(1)
    fire_gathers(1)

    @pl.loop(0, NCHUNK, step=2)
    def _fm(c):
        pre(c, 0)

        @pl.when(c + 2 < NCHUNK)
        def _():
            fire_idx(c + 2, 0)

        main(c, 0)

        @pl.when(c + 2 < NCHUNK)
        def _():
            wait_idx(0)
            fire_gathers(0)

        pre(c + 1, 1)

        @pl.when(c + 3 < NCHUNK)
        def _():
            fire_idx(c + 3, 1)

        main(c + 1, 1)

        @pl.when(c + 3 < NCHUNK)
        def _():
            wait_idx(1)
            fire_gathers(1)

    for buf in range(2):
        pltpu.make_async_copy(
            outvs[buf], cross_h.at[pl.ds(0, CHUNK), pl.ds(j16, PLEN)],
            osems[buf]).wait()


def _sc_lin_body(gidxT_h, linw_h, lin_h, lidxv, linrowv, loutv, sem):
    ci = lax.axis_index("c")
    si = lax.axis_index("s")
    wid = si * NSC + ci
    pltpu.sync_copy(gidxT_h.at[wid], lidxv)
    lcps = [
        pltpu.async_copy(linw_h.at[lidxv.at[f]],
                         linrowv.at[pl.ds(f * 128, 128)], sem)
        for f in range(F)
    ]
    for cp in lcps:
        cp.wait()
    for g in range(LROWS // 16):
        acc0 = jnp.zeros((16,), jnp.float32)
        acc1 = jnp.zeros((16,), jnp.float32)
        for f in range(F):
            v = linrowv[pl.ds(f * 128 + g * 16, 16)]
            if f % 2 == 0:
                acc0 = acc0 + v
            else:
                acc1 = acc1 + v
        loutv[pl.ds(g * 16, 16)] = acc0 + acc1
    pltpu.sync_copy(loutv, lin_h.at[pl.ds(wid * LROWS, LROWS)])


_SC_PARAMS = pltpu.CompilerParams(use_tc_tiling_on_sc=False,
                                  needs_layout_passes=False,
                                  disable_bounds_checks=True)


def _sc_fm_wrap(gidx_h, cb_h, cbtail_h, tbl_h, cross_h,
                tblv, idxv0, idxv1, jidxv0, jidxv1, codesv0, codesv1,
                tailv, outv0, outv1, gsem0, gsem1, isem0, isem1,
                osem0, osem1):
    _sc_fm_body(gidx_h, cb_h, cbtail_h, tbl_h, cross_h,
                tblv, (idxv0, idxv1), (jidxv0, jidxv1),
                (codesv0, codesv1), tailv, (outv0, outv1),
                (gsem0, gsem1), (isem0, isem1), (osem0, osem1))


@jax.jit
def _sc_call(gidx, gidxT, cbm, cbtail, tbl, linw):
    fm_k = functools.partial(
        pl.kernel,
        out_type=jax.ShapeDtypeStruct((B, DIM), jnp.float32),
        mesh=_mesh,
        scratch_types=[
            pltpu.VMEM((F * K * PLEN,), jnp.float32),
            pltpu.VMEM((CROWS, 128), jnp.int32),
            pltpu.VMEM((CROWS, 128), jnp.int32),
            pltpu.VMEM((CROWS, 128), jnp.int32),
            pltpu.VMEM((CROWS, 128), jnp.int32),
            pltpu.VMEM((CHUNK * F,), jnp.int32),
            pltpu.VMEM((CHUNK * F,), jnp.int32),
            pltpu.VMEM((NTAIL * M,), jnp.int32),
            pltpu.VMEM((CHUNK, PLEN), jnp.float32),
            pltpu.VMEM((CHUNK, PLEN), jnp.float32),
            pltpu.SemaphoreType.DMA,
            pltpu.SemaphoreType.DMA,
            pltpu.SemaphoreType.DMA,
            pltpu.SemaphoreType.DMA,
            pltpu.SemaphoreType.DMA,
            pltpu.SemaphoreType.DMA,
        ],
        compiler_params=_SC_PARAMS,
    )(_sc_fm_wrap)
    cross = fm_k(gidx, cbm, cbtail, tbl)
    linw, cross = lax.optimization_barrier((linw, cross))
    lin = functools.partial(
        pl.kernel,
        out_type=jax.ShapeDtypeStruct((B,), jnp.float32),
        mesh=_mesh,
        scratch_types=[
            pltpu.VMEM((F, 128), jnp.int32),
            pltpu.VMEM((F * 128,), jnp.float32),
            pltpu.VMEM((LROWS,), jnp.float32),
            pltpu.SemaphoreType.DMA,
        ],
        compiler_params=_SC_PARAMS,
    )(_sc_lin_body)(gidxT, linw)
    return cross, lin


def _lw_body(in_ref, out_ref):
    out_ref[...] = in_ref[0]


def _lw_extract(lwT):
    blk = 65536
    return pl.pallas_call(
        _lw_body,
        grid=(pl.cdiv(TOTAL, blk),),
        in_specs=[pl.BlockSpec((1, blk), lambda i: (0, i))],
        out_specs=pl.BlockSpec((blk,), lambda i: (i,)),
        out_shape=jax.ShapeDtypeStruct((TOTAL,), jnp.float32),
    )(lwT)


def _mlp_body(cross_ref, lin_ref, w1_ref, b1_ref, w2_ref, b2_ref,
              w3_ref, b3_ref, out_ref):
    h = jnp.dot(cross_ref[...], w1_ref[...],
                preferred_element_type=jnp.float32) + b1_ref[...]
    h = jnp.maximum(h, 0.0)
    h = jnp.dot(h, w2_ref[...],
                preferred_element_type=jnp.float32) + b2_ref[...]
    h = jnp.maximum(h, 0.0)
    o = jnp.dot(h, w3_ref[...], preferred_element_type=jnp.float32)
    out_ref[...] = o + b3_ref[...] + lin_ref[...]


def _mlp_call(cross, lin2, w1f, beta1, w2f, beta2, w3f, beta3):
    bb = 512
    return pl.pallas_call(
        _mlp_body,
        grid=(B // bb,),
        in_specs=[
            pl.BlockSpec((bb, DIM), lambda i: (i, 0)),
            pl.BlockSpec((bb, 1), lambda i: (i, 0)),
            pl.BlockSpec((DIM, 1024), lambda i: (0, 0)),
            pl.BlockSpec((1, 1024), lambda i: (0, 0)),
            pl.BlockSpec((1024, 512), lambda i: (0, 0)),
            pl.BlockSpec((1, 512), lambda i: (0, 0)),
            pl.BlockSpec((512, 1), lambda i: (0, 0)),
            pl.BlockSpec((1, 1), lambda i: (0, 0)),
        ],
        out_specs=pl.BlockSpec((bb, 1), lambda i: (i, 0)),
        out_shape=jax.ShapeDtypeStruct((B, 1), jnp.float32),
    )(cross, lin2, w1f, beta1, w2f, beta2, w3f, beta3)


_OFFS = np.concatenate([[0], np.cumsum([100000] * F)[:-1]]).astype(np.int32)


def kernel(x, cb_index, codebooks, linear_w, linear_b, bn0_g, bn0_b,
           w1, b1, g1, be1, w2, b2, g2, be2, w3, b3):
    gflat = (x + jnp.asarray(_OFFS)[None, :]).astype(jnp.int32)
    pbase = ((gflat >> 7) * 1024 + (gflat & 127)).astype(jnp.int32)
    gidx = pbase.reshape(B * F // 128, 128)
    gidxT = gflat.T.reshape(F, NW, LROWS).transpose(1, 0, 2)
    tbl = (codebooks.T.reshape(16, 8, 52, 128)
           .transpose(0, 2, 1, 3).reshape(M, F * K * PLEN))
    cbm = (cb_index[:NMAIN].reshape(NMAIN // 128, 128, M)
           .transpose(0, 2, 1).reshape(-1))
    cbtail = cb_index[NMAIN:].reshape(-1)
    lw_b, cbm = lax.optimization_barrier((linear_w.T, cbm))
    linw = _lw_extract(lw_b)

    cross, lin = _sc_call(gidx, gidxT, cbm, cbtail, tbl, linw)

    c = 1.0 / jnp.sqrt(jnp.float32(1.0 + EPS))
    w1f = (w1 * (c * bn0_g)[None, :]).T * (c * g1)[None, :]
    beta1 = (w1 @ bn0_b + b1) * (c * g1) + be1
    w2f = w2.T * (c * g2)[None, :]
    beta2 = b2 * (c * g2) + be2
    w3f = w3.T
    beta3 = (b3 + linear_b).reshape(1, 1)

    out = _mlp_call(cross, lin.reshape(B, 1), w1f, beta1.reshape(1, -1),
                    w2f, beta2.reshape(1, -1), w3f, beta3)
    return out.reshape(B)


# speedup vs baseline: 1.0296x; 1.0296x over previous
import functools

import jax
import jax.numpy as jnp
import numpy as np
from jax import lax
from jax.experimental import pallas as pl
from jax.experimental.pallas import tpu as pltpu
from jax.experimental.pallas import tpu_sc as plsc

F = 26
DIM = 128
M = 8
K = 256
B = 4096
PLEN = DIM // M
TOTAL = 100000 * F
EPS = 1e-5

NSC = 2
NSUB = 16
NW = NSC * NSUB
QROWS = B // 4
CHUNK = 64
NCHUNK = QROWS // CHUNK
CROWS = CHUNK * F // 128
LROWS = B // NW

_mesh = plsc.VectorSubcoreMesh(core_axis_name="c", subcore_axis_name="s")


NMAIN = (TOTAL // 128) * 128
NTAIL = TOTAL - NMAIN
WMAX = NMAIN * M


def _fire_gathers(cb_h, jidxb, codesb, sem):
    return [
        pltpu.async_copy(cb_h.at[jidxb.at[i]],
                         codesb.at[pl.ds(i * 128, 128)], sem)
        for i in range(CROWS)
    ]


def _sc_fm_body(gidx_h, cb_h, cbtail_h, tbl_h, cross_h,
                tblv, idxvs, jidxvs, codesvs, tailv, outvs,
                gsems, isems, osems):
    ci = lax.axis_index("c")
    si = lax.axis_index("s")
    wid = si * NSC + ci
    j = lax.rem(wid, M)
    q = wid // M
    j16 = j * PLEN

    pltpu.sync_copy(tbl_h.at[j], tblv)
    pltpu.sync_copy(cbtail_h, tailv)

    iota = lax.iota(jnp.int32, 16)
    zeros_i = jnp.zeros((16,), jnp.int32)
    j128 = zeros_i + j * 128
    jv = zeros_i + j
    rbase = q * (QROWS * F // 128)

    def fire_idx(c, buf):
        pltpu.async_copy(gidx_h.at[pl.ds(rbase + c * CROWS, CROWS)],
                         idxvs[buf], isems[buf])

    def wait_idx(buf):
        pltpu.make_async_copy(gidx_h.at[pl.ds(rbase, CROWS)],
                              idxvs[buf], isems[buf]).wait()

    def fire_gathers(buf):
        for r in range(CROWS):
            for l in range(8):
                s = pl.ds(l * 16, 16)
                w = idxvs[buf][r, s] + j128
                jidxvs[buf][r, s] = jnp.minimum(w, WMAX - 1)
        _fire_gathers(cb_h, jidxvs[buf], codesvs[buf], gsems[buf])

    def pre(c, buf):
        for i in range(CROWS):
            pltpu.make_async_copy(cb_h.at[jidxvs[buf].at[i]],
                                  codesvs[buf].at[pl.ds(i * 128, 128)],
                                  gsems[buf]).wait()
        idxb, codesb = idxvs[buf], codesvs[buf]
        for gb in range(CHUNK // 16):
            p25 = (gb * 16 + iota) * F + (F - 1)
            pb = plsc.load_gather(
                idxb, [lax.shift_right_logical(p25, 7),
                       lax.bitwise_and(p25, 127)])
            is_tail = pb >= WMAX
            tv = plsc.load_gather(
                tailv, [jnp.clip(pb - WMAX, 0, NTAIL - 1) * M + jv])
            cur = plsc.load_gather(codesb, [p25])
            plsc.store_scatter(codesb, [p25], jnp.where(is_tail, tv, cur))

    def main(c, buf):
        codesb, outv = codesvs[buf], outvs[buf]

        @pl.when(c >= 2)
        def _():
            pltpu.make_async_copy(
                outvs[buf], cross_h.at[pl.ds(0, CHUNK), pl.ds(j16, PLEN)],
                osems[buf]).wait()

        @plsc.parallel_loop(0, CHUNK // 16)
        def _grp(gb):
            rowbase = (gb * 16 + iota) * F
            acc_s = [jnp.zeros((16,), jnp.float32) for _ in range(PLEN)]
            acc_t = [jnp.zeros((16,), jnp.float32) for _ in range(PLEN)]
            for f in range(F):
                codes = plsc.load_gather(codesb, [rowbase + f])
                ch = (lax.shift_left(lax.shift_right_logical(codes, 7), 10)
                      + lax.bitwise_and(codes, 127))
                for dd in range(PLEN):
                    k1 = ((dd >> 3) * 52 + f * 2) * 1024 + (dd & 7) * 128
                    v = plsc.load_gather(tblv, [ch + k1])
                    acc_s[dd] = acc_s[dd] + v
                    acc_t[dd] = acc_t[dd] + v * v
            orow = gb * 16 + iota
            for dd in range(PLEN):
                cv = 0.5 * (acc_s[dd] * acc_s[dd] - acc_t[dd])
                plsc.store_scatter(outv, [orow, zeros_i + dd], cv)

        b0 = q * QROWS + c * CHUNK
        pltpu.async_copy(outv, cross_h.at[pl.ds(b0, CHUNK), pl.ds(j16, PLEN)],
                         osems[buf])

    fire_idx(0, 0)
    fire_idx(1, 1)
    wait_idx(0)
    fire_gathers(0)
    wait_idx(1)
    fire_gathers(1)

    @pl.loop(0, NCHUNK, step=2)
    def _fm(c):
        pre(c, 0)

        @pl.when(c + 2 < NCHUNK)
        def _():
            fire_idx(c + 2, 0)

        main(c, 0)

        @pl.when(c + 2 < NCHUNK)
        def _():
            wait_idx(0)
            fire_gathers(0)

        pre(c + 1, 1)

        @pl.when(c + 3 < NCHUNK)
        def _():
            fire_idx(c + 3, 1)

        main(c + 1, 1)

        @pl.when(c + 3 < NCHUNK)
        def _():
            wait_idx(1)
            fire_gathers(1)

    for buf in range(2):
        pltpu.make_async_copy(
            outvs[buf], cross_h.at[pl.ds(0, CHUNK), pl.ds(j16, PLEN)],
            osems[buf]).wait()


def _sc_lin_body(gidxT_h, linw_h, lin_h, lidxv, linrowv, loutv, sem):
    ci = lax.axis_index("c")
    si = lax.axis_index("s")
    wid = si * NSC + ci
    pltpu.sync_copy(gidxT_h.at[wid], lidxv)
    lcps = [
        pltpu.async_copy(linw_h.at[lidxv.at[f]],
                         linrowv.at[pl.ds(f * 128, 128)], sem)
        for f in range(F)
    ]
    for cp in lcps:
        cp.wait()
    for g in range(LROWS // 16):
        acc0 = jnp.zeros((16,), jnp.float32)
        acc1 = jnp.zeros((16,), jnp.float32)
        for f in range(F):
            v = linrowv[pl.ds(f * 128 + g * 16, 16)]
            if f % 2 == 0:
                acc0 = acc0 + v
            else:
                acc1 = acc1 + v
        loutv[pl.ds(g * 16, 16)] = acc0 + acc1
    pltpu.sync_copy(loutv, lin_h.at[pl.ds(wid * LROWS, LROWS)])


_SC_PARAMS = pltpu.CompilerParams(use_tc_tiling_on_sc=False,
                                  needs_layout_passes=False,
                                  disable_bounds_checks=True)


def _sc_fm_wrap(gidx_h, cb_h, cbtail_h, tbl_h, cross_h,
                tblv, idxv0, idxv1, jidxv0, jidxv1, codesv0, codesv1,
                tailv, outv0, outv1, gsem0, gsem1, isem0, isem1,
                osem0, osem1):
    _sc_fm_body(gidx_h, cb_h, cbtail_h, tbl_h, cross_h,
                tblv, (idxv0, idxv1), (jidxv0, jidxv1),
                (codesv0, codesv1), tailv, (outv0, outv1),
                (gsem0, gsem1), (isem0, isem1), (osem0, osem1))


@jax.jit
def _sc_call(gidx, gidxT, cbm, cbtail, tbl, linw):
    fm_k = functools.partial(
        pl.kernel,
        out_type=jax.ShapeDtypeStruct((B, DIM), jnp.float32),
        mesh=_mesh,
        scratch_types=[
            pltpu.VMEM((F * K * PLEN,), jnp.float32),
            pltpu.VMEM((CROWS, 128), jnp.int32),
            pltpu.VMEM((CROWS, 128), jnp.int32),
            pltpu.VMEM((CROWS, 128), jnp.int32),
            pltpu.VMEM((CROWS, 128), jnp.int32),
            pltpu.VMEM((CHUNK * F,), jnp.int32),
            pltpu.VMEM((CHUNK * F,), jnp.int32),
            pltpu.VMEM((NTAIL * M,), jnp.int32),
            pltpu.VMEM((CHUNK, PLEN), jnp.float32),
            pltpu.VMEM((CHUNK, PLEN), jnp.float32),
            pltpu.SemaphoreType.DMA,
            pltpu.SemaphoreType.DMA,
            pltpu.SemaphoreType.DMA,
            pltpu.SemaphoreType.DMA,
            pltpu.SemaphoreType.DMA,
            pltpu.SemaphoreType.DMA,
        ],
        compiler_params=_SC_PARAMS,
    )(_sc_fm_wrap)
    cross = fm_k(gidx, cbm, cbtail, tbl)
    linw, cross = lax.optimization_barrier((linw, cross))
    lin = functools.partial(
        pl.kernel,
        out_type=jax.ShapeDtypeStruct((B,), jnp.float32),
        mesh=_mesh,
        scratch_types=[
            pltpu.VMEM((F, 128), jnp.int32),
            pltpu.VMEM((F * 128,), jnp.float32),
            pltpu.VMEM((LROWS,), jnp.float32),
            pltpu.SemaphoreType.DMA,
        ],
        compiler_params=_SC_PARAMS,
    )(_sc_lin_body)(gidxT, linw)
    return cross, lin


def _lw_body(in_ref, out_ref):
    out_ref[...] = in_ref[0]


def _lw_extract(lwT):
    blk = 65536
    return pl.pallas_call(
        _lw_body,
        grid=(pl.cdiv(TOTAL, blk),),
        in_specs=[pl.BlockSpec((1, blk), lambda i: (0, i))],
        out_specs=pl.BlockSpec((blk,), lambda i: (i,)),
        out_shape=jax.ShapeDtypeStruct((TOTAL,), jnp.float32),
    )(lwT)


def _mlp_body(cross_ref, lin_ref, w1_ref, b1_ref, w2_ref, b2_ref,
              w3_ref, b3_ref, out_ref):
    h = jnp.dot(cross_ref[...], w1_ref[...],
                preferred_element_type=jnp.float32) + b1_ref[...]
    h = jnp.maximum(h, 0.0)
    h = jnp.dot(h, w2_ref[...],
                preferred_element_type=jnp.float32) + b2_ref[...]
    h = jnp.maximum(h, 0.0)
    o = jnp.dot(h, w3_ref[...], preferred_element_type=jnp.float32)
    out_ref[...] = o + b3_ref[...] + lin_ref[...]


def _mlp_call(cross, lin2, w1f, beta1, w2f, beta2, w3f, beta3):
    bb = 512
    return pl.pallas_call(
        _mlp_body,
        grid=(B // bb,),
        in_specs=[
            pl.BlockSpec((bb, DIM), lambda i: (i, 0)),
            pl.BlockSpec((bb, 1), lambda i: (i, 0)),
            pl.BlockSpec((DIM, 1024), lambda i: (0, 0)),
            pl.BlockSpec((1, 1024), lambda i: (0, 0)),
            pl.BlockSpec((1024, 512), lambda i: (0, 0)),
            pl.BlockSpec((1, 512), lambda i: (0, 0)),
            pl.BlockSpec((512, 1), lambda i: (0, 0)),
            pl.BlockSpec((1, 1), lambda i: (0, 0)),
        ],
        out_specs=pl.BlockSpec((bb, 1), lambda i: (i, 0)),
        out_shape=jax.ShapeDtypeStruct((B, 1), jnp.float32),
    )(cross, lin2, w1f, beta1, w2f, beta2, w3f, beta3)


_OFFS = np.concatenate([[0], np.cumsum([100000] * F)[:-1]]).astype(np.int32)


def kernel(x, cb_index, codebooks, linear_w, linear_b, bn0_g, bn0_b,
           w1, b1, g1, be1, w2, b2, g2, be2, w3, b3):
    gflat = (x + jnp.asarray(_OFFS)[None, :]).astype(jnp.int32)
    pbase = ((gflat >> 7) * 1024 + (gflat & 127)).astype(jnp.int32)
    gidx = pbase.reshape(B * F // 128, 128)
    gidxT = gflat.T.reshape(F, NW, LROWS).transpose(1, 0, 2)
    tbl = (codebooks.T.reshape(16, 8, 52, 128)
           .transpose(0, 2, 1, 3).reshape(M, F * K * PLEN))
    cbm = (cb_index[:NMAIN].reshape(NMAIN // 128, 128, M)
           .transpose(0, 2, 1).reshape(-1))
    cbtail = cb_index[NMAIN:].reshape(-1)
    lw_b, cbm = lax.optimization_barrier((linear_w.T, cbm))
    linw = _lw_extract(lw_b)

    cross, lin = _sc_call(gidx, gidxT, cbm, cbtail, tbl, linw)

    c = 1.0 / jnp.sqrt(jnp.float32(1.0 + EPS))
    w1f = (w1 * (c * bn0_g)[None, :]).T * (c * g1)[None, :]
    beta1 = (w1 @ bn0_b + b1) * (c * g1) + be1
    w2f = w2.T * (c * g2)[None, :]
    beta2 = b2 * (c * g2) + be2
    w3f = w3.T
    beta3 = (b3 + linear_b).reshape(1, 1)

    out = _mlp_call(cross, lin.reshape(B, 1), w1f, beta1.reshape(1, -1),
                    w2f, beta2.reshape(1, -1), w3f, beta3)
    return out.reshape(B)
